# Initial kernel scaffold; baseline (speedup 1.0000x reference)
import jax
import jax.numpy as jnp
from jax.experimental import pallas as pl


def kernel(x1, x2, params, edge_index1, edge_index2, jg_edge_index):
    raise NotImplementedError("write your pallas kernel here")


# speedup vs baseline: 1.0675x; 1.0675x over previous
import functools
import jax
import jax.numpy as jnp
from jax.experimental import pallas as pl
from jax.experimental.pallas import tpu as pltpu

N = 10000
DIM = 128
H = 8
DH = DIM // H


def _mlp2_body(x_ref, w1_ref, b1_ref, w2_ref, b2_ref, o_ref):
    u = jnp.maximum(
        jnp.dot(x_ref[...], w1_ref[...], preferred_element_type=jnp.float32)
        + b1_ref[...], 0.0)
    o_ref[...] = (
        jnp.dot(u, w2_ref[...], preferred_element_type=jnp.float32)
        + b2_ref[...])


def _mlp2_pallas(x, w1, b1, w2, b2, block_rows):
    rows, k = x.shape
    d1 = w1.shape[1]
    d2 = w2.shape[1]
    grid = rows // block_rows
    return pl.pallas_call(
        _mlp2_body,
        grid=(grid,),
        in_specs=[
            pl.BlockSpec((block_rows, k), lambda i: (i, 0)),
            pl.BlockSpec((k, d1), lambda i: (0, 0)),
            pl.BlockSpec((1, d1), lambda i: (0, 0)),
            pl.BlockSpec((d1, d2), lambda i: (0, 0)),
            pl.BlockSpec((1, d2), lambda i: (0, 0)),
        ],
        out_specs=pl.BlockSpec((block_rows, d2), lambda i: (i, 0)),
        out_shape=jax.ShapeDtypeStruct((rows, d2), jnp.float32),
    )(x, w1, b1, w2, b2)


def _pair_body(p_ref, w1_ref, b1_ref, w2_ref, b2_ref, w3_ref, b3_ref, o_ref):
    h = jnp.maximum(
        jnp.dot(p_ref[...], w1_ref[...], preferred_element_type=jnp.float32)
        + b1_ref[...], 0.0)
    h = jnp.maximum(
        jnp.dot(h, w2_ref[...], preferred_element_type=jnp.float32)
        + b2_ref[...], 0.0)
    o_ref[...] = (
        jnp.dot(h, w3_ref[...], preferred_element_type=jnp.float32)
        + b3_ref[...])


def _pair_pallas(p, w1, b1, w2, b2, w3, b3, block_rows):
    rows, k = p.shape
    grid = rows // block_rows
    return pl.pallas_call(
        _pair_body,
        grid=(grid,),
        in_specs=[
            pl.BlockSpec((block_rows, k), lambda i: (i, 0)),
            pl.BlockSpec((k, 128), lambda i: (0, 0)),
            pl.BlockSpec((1, 128), lambda i: (0, 0)),
            pl.BlockSpec((128, 128), lambda i: (0, 0)),
            pl.BlockSpec((1, 128), lambda i: (0, 0)),
            pl.BlockSpec((128, 1), lambda i: (0, 0)),
            pl.BlockSpec((1, 1), lambda i: (0, 0)),
        ],
        out_specs=pl.BlockSpec((block_rows, 1), lambda i: (i, 0)),
        out_shape=jax.ShapeDtypeStruct((rows, 1), jnp.float32),
    )(p, w1, b1, w2, b2, w3, b3)


def _gatv2(x, src, dst, Wl, Wr, a, b):
    n = x.shape[0]
    xl = (x @ Wl).reshape(n, H, DH)
    xr = (x @ Wr).reshape(n, H, DH)
    m = jax.nn.leaky_relu(xl[src] + xr[dst], 0.2)
    e = (m * a[None, :, :]).sum(-1)
    ee = jnp.exp(e)
    denom = jax.ops.segment_sum(ee, dst, num_segments=n)
    num = jax.ops.segment_sum(ee[:, :, None] * xl[src], dst, num_segments=n)
    out = num / (denom[:, :, None] + 1e-16)
    return out.reshape(n, H * DH) + b


def _bn(x, g, b):
    mu = x.mean(0)
    var = x.var(0)
    return (x - mu) / jnp.sqrt(var + 1e-5) * g + b


def _gnn(x, src, dst, p):
    x = _gatv2(x, src, dst, p['g1Wl'], p['g1Wr'], p['g1a'], p['g1b'])
    x = _bn(x, p['bng'], p['bnb'])
    x = jax.nn.elu(x)
    x = _gatv2(x, src, dst, p['g2Wl'], p['g2Wr'], p['g2a'], p['g2b'])
    return x


def kernel(x1, x2, params, edge_index1, edge_index2, jg_edge_index):
    p = params
    n = x1.shape[0]
    loop = jnp.arange(n, dtype=edge_index1.dtype)
    s1 = jnp.concatenate([edge_index1[0], loop])
    d1 = jnp.concatenate([edge_index1[1], loop])
    s2 = jnp.concatenate([edge_index2[0], loop])
    d2 = jnp.concatenate([edge_index2[1], loop])

    w1c = jnp.concatenate([p['fW1'], p['eW1']], axis=1)
    b1c = jnp.concatenate([p['fb1'], p['eb1']])[None, :]
    w2s = jnp.concatenate([p['fW2'], p['eW2']], axis=0)
    b2s = (p['fb2'] + p['eb2'])[None, :]

    xcat = jnp.concatenate([x1, x2], axis=0)
    hcat = _mlp2_pallas(xcat, w1c, b1c, w2s, b2s, block_rows=400)
    h1, h2 = hcat[:n], hcat[n:]

    h1 = _gnn(h1, s1, d1, p)
    h2 = _gnn(h2, s2, d2, p)

    pair = jnp.concatenate([h1[jg_edge_index[0]], h2[jg_edge_index[1]]],
                           axis=1)
    return _pair_pallas(pair, p['pW1'], p['pb1'][None, :],
                        p['pW2'], p['pb2'][None, :],
                        p['pW3'], p['pb3'][None, :], block_rows=800)


# speedup vs baseline: 16.7973x; 15.7352x over previous
import functools
import jax
import jax.numpy as jnp
from jax import lax
from jax.experimental import pallas as pl
from jax.experimental.pallas import tpu as pltpu
from jax.experimental.pallas import tpu_sc as plsc

N = 10000
DIM = 128
H = 8
DH = DIM // H
NPAD = 10240
NT = 2 * NPAD
PADROW = 10100
CH = 128
HHALF = 4
VW = HHALF * DH
ACCW = VW + 16
NSUB = 16


def _mlp2_body(x_ref, w1_ref, b1_ref, w2_ref, b2_ref, o_ref):
    u = jnp.maximum(
        jnp.dot(x_ref[...], w1_ref[...], preferred_element_type=jnp.float32)
        + b1_ref[...], 0.0)
    o_ref[...] = (
        jnp.dot(u, w2_ref[...], preferred_element_type=jnp.float32)
        + b2_ref[...])


def _mlp2_pallas(x, w1, b1, w2, b2, block_rows):
    rows, k = x.shape
    d1 = w1.shape[1]
    d2 = w2.shape[1]
    return pl.pallas_call(
        _mlp2_body,
        grid=(rows // block_rows,),
        in_specs=[
            pl.BlockSpec((block_rows, k), lambda i: (i, 0)),
            pl.BlockSpec((k, d1), lambda i: (0, 0)),
            pl.BlockSpec((1, d1), lambda i: (0, 0)),
            pl.BlockSpec((d1, d2), lambda i: (0, 0)),
            pl.BlockSpec((1, d2), lambda i: (0, 0)),
        ],
        out_specs=pl.BlockSpec((block_rows, d2), lambda i: (i, 0)),
        out_shape=jax.ShapeDtypeStruct((rows, d2), jnp.float32),
    )(x, w1, b1, w2, b2)


def _proj_body(x_ref, wl_ref, wr_ref, la_ref, lb_ref, ra_ref, rb_ref):
    x = x_ref[...]
    xl = jnp.dot(x, wl_ref[...], preferred_element_type=jnp.float32)
    xr = jnp.dot(x, wr_ref[...], preferred_element_type=jnp.float32)
    la_ref[...] = xl[:, :VW]
    lb_ref[...] = xl[:, VW:]
    ra_ref[...] = xr[:, :VW]
    rb_ref[...] = xr[:, VW:]


def _proj_pallas(x, wl, wr, block_rows):
    rows = x.shape[0]
    half = jax.ShapeDtypeStruct((rows, VW), jnp.float32)
    return pl.pallas_call(
        _proj_body,
        grid=(rows // block_rows,),
        in_specs=[
            pl.BlockSpec((block_rows, DIM), lambda i: (i, 0)),
            pl.BlockSpec((DIM, DIM), lambda i: (0, 0)),
            pl.BlockSpec((DIM, DIM), lambda i: (0, 0)),
        ],
        out_specs=[pl.BlockSpec((block_rows, VW), lambda i: (i, 0))] * 4,
        out_shape=[half, half, half, half],
    )(x, wl, wr)


def _edge_kernel_body(xl_hbm, xr_hbm, src_hbm, dst_hbm, a_hbm, out_hbm,
                      sidx, didx, xlg, xrg, val, av, acc, sem, sem2):
    cid = lax.axis_index("c")
    sid = lax.axis_index("s")
    ept = src_hbm.shape[1] // NSUB
    nchunk = ept // CH
    tile_base = sid * ept
    goff = cid * NPAD

    pltpu.sync_copy(a_hbm, av)

    def zrow(r, _):
        for c in range(ACCW // 16):
            val[r, pl.ds(c * 16, 16)] = jnp.zeros((16,), jnp.float32)
        return 0
    lax.fori_loop(0, CH, zrow, 0)
    rows_per_tile = NPAD // NSUB
    for z in range(rows_per_tile // CH):
        pltpu.sync_copy(val, acc.at[pl.ds(sid * rows_per_tile + z * CH, CH)])
    plsc.subcore_barrier()

    lane = lax.iota(jnp.int32, 16)

    def chunk(k, _):
        base = tile_base + k * CH
        pltpu.sync_copy(src_hbm.at[cid, pl.ds(base, CH)], sidx)
        pltpu.sync_copy(dst_hbm.at[cid, pl.ds(base, CH)], didx)
        cp1 = pltpu.async_copy(xl_hbm.at[sidx], xlg, sem)
        cp2 = pltpu.async_copy(xr_hbm.at[didx], xrg, sem2)
        cp1.wait()
        cp2.wait()

        def edge(ed, _):
            wlanes = jnp.zeros((16,), jnp.float32)
            for h in range(HHALF):
                xlv = xlg[ed, pl.ds(h * DH, DH)]
                xrv = xrg[ed, pl.ds(h * DH, DH)]
                s = xlv + xrv
                m = jnp.where(s > 0, s, 0.2 * s)
                t = m * av[pl.ds(h * DH, DH)]
                e = jnp.sum(t)
                wv = jnp.exp(jnp.full((16,), e, jnp.float32))
                val[ed, pl.ds(h * DH, DH)] = wv * xlv
                wlanes = jnp.where(lane == h, wv, wlanes)
            val[ed, pl.ds(VW, 16)] = wlanes
            return 0
        lax.fori_loop(0, CH, edge, 0)

        for c in range(CH // 16):
            didx[pl.ds(c * 16, 16)] = didx[pl.ds(c * 16, 16)] - goff
        pltpu.sync_copy(val, acc.at[didx], add=True)
        return 0
    lax.fori_loop(0, nchunk, chunk, 0)

    plsc.subcore_barrier()
    for z in range(rows_per_tile // CH):
        r0 = sid * rows_per_tile + z * CH
        pltpu.sync_copy(acc.at[pl.ds(r0, CH)],
                        out_hbm.at[cid, pl.ds(r0, CH)])


def _edge_pass(xl, xr, srcp, dstp, a_half):
    mesh = plsc.VectorSubcoreMesh(core_axis_name="c", subcore_axis_name="s")
    kfn = functools.partial(
        pl.kernel,
        mesh=mesh,
        compiler_params=pltpu.CompilerParams(needs_layout_passes=False,
                                             use_tc_tiling_on_sc=False),
        out_type=jax.ShapeDtypeStruct((2, NPAD, ACCW), jnp.float32),
        scratch_types=[
            pltpu.VMEM((CH,), jnp.int32),
            pltpu.VMEM((CH,), jnp.int32),
            pltpu.VMEM((CH, VW), jnp.float32),
            pltpu.VMEM((CH, VW), jnp.float32),
            pltpu.VMEM((CH, ACCW), jnp.float32),
            pltpu.VMEM((VW,), jnp.float32),
            pltpu.VMEM_SHARED((NPAD, ACCW), jnp.float32),
            pltpu.SemaphoreType.DMA,
            pltpu.SemaphoreType.DMA,
        ],
    )(_edge_kernel_body)
    return kfn(xl, xr, srcp, dstp, a_half)


def _pair_gather_body(h_hbm, i0_hbm, i1_hbm, xs_hbm, xt_hbm,
                      idx0, idx1, b0, b1, sem, sem2):
    cid = lax.axis_index("c")
    sid = lax.axis_index("s")
    wid = sid * 2 + cid
    ept = i0_hbm.shape[0] // 32
    nchunk = ept // CH
    tile_base = wid * ept

    def chunk(k, _):
        base = tile_base + k * CH
        pltpu.sync_copy(i0_hbm.at[pl.ds(base, CH)], idx0)
        pltpu.sync_copy(i1_hbm.at[pl.ds(base, CH)], idx1)
        cp1 = pltpu.async_copy(h_hbm.at[idx0], b0, sem)
        cp2 = pltpu.async_copy(h_hbm.at[idx1], b1, sem2)
        cp1.wait()
        cp2.wait()
        pltpu.sync_copy(b0, xs_hbm.at[pl.ds(base, CH)])
        pltpu.sync_copy(b1, xt_hbm.at[pl.ds(base, CH)])
        return 0
    lax.fori_loop(0, nchunk, chunk, 0)


def _pair_gather(h_table, i0, i1):
    ejp = i0.shape[0]
    mesh = plsc.VectorSubcoreMesh(core_axis_name="c", subcore_axis_name="s")
    kfn = functools.partial(
        pl.kernel,
        mesh=mesh,
        compiler_params=pltpu.CompilerParams(needs_layout_passes=False,
                                             use_tc_tiling_on_sc=False),
        out_type=[jax.ShapeDtypeStruct((ejp, DIM), jnp.float32),
                  jax.ShapeDtypeStruct((ejp, DIM), jnp.float32)],
        scratch_types=[
            pltpu.VMEM((CH,), jnp.int32),
            pltpu.VMEM((CH,), jnp.int32),
            pltpu.VMEM((CH, DIM), jnp.float32),
            pltpu.VMEM((CH, DIM), jnp.float32),
            pltpu.SemaphoreType.DMA,
            pltpu.SemaphoreType.DMA,
        ],
    )(_pair_gather_body)
    return kfn(h_table, i0, i1)


def _half_norm(accs, expand_ref, gb_ref, lo):
    num = accs[:, :VW]
    den = accs[:, VW:VW + HHALF]
    denw = jnp.dot(den, expand_ref[...], preferred_element_type=jnp.float32)
    return num / (denw + 1e-16) + gb_ref[0, lo:lo + VW]


def _norm_body_bn(acca_ref, accb_ref, exp_ref, gb_ref, g_ref, b_ref, o_ref):
    for i, acc_ref in enumerate((acca_ref, accb_ref)):
        lo = i * VW
        x = _half_norm(acc_ref[0], exp_ref, gb_ref, lo)
        xr = x[:N, :]
        mu = jnp.mean(xr, axis=0, keepdims=True)
        var = jnp.mean(xr * xr, axis=0, keepdims=True) - mu * mu
        y = ((xr - mu) / jnp.sqrt(var + 1e-5) * g_ref[0, lo:lo + VW]
             + b_ref[0, lo:lo + VW])
        y = jnp.where(y > 0, y, jnp.exp(jnp.minimum(y, 0.0)) - 1.0)
        o_ref[0, :N, pl.ds(lo, VW)] = y
        o_ref[0, N:, pl.ds(lo, VW)] = jnp.zeros((NPAD - N, VW), jnp.float32)


def _norm_body(acca_ref, accb_ref, exp_ref, gb_ref, o_ref):
    for i, acc_ref in enumerate((acca_ref, accb_ref)):
        lo = i * VW
        x = _half_norm(acc_ref[0], exp_ref, gb_ref, lo)
        o_ref[0, :N, pl.ds(lo, VW)] = x[:N, :]
        o_ref[0, N:, pl.ds(lo, VW)] = jnp.zeros((NPAD - N, VW), jnp.float32)


def _norm_pallas(acca, accb, expand, gbias, bn_g=None, bn_b=None):
    acc_spec = pl.BlockSpec((1, NPAD, ACCW), lambda g: (g, 0, 0))
    full_spec = pl.BlockSpec((1, DIM), lambda g: (0, 0))
    if bn_g is not None:
        body = _norm_body_bn
        args = (acca, accb, expand, gbias, bn_g, bn_b)
        in_specs = [acc_spec, acc_spec,
                    pl.BlockSpec((HHALF, VW), lambda g: (0, 0)),
                    full_spec, full_spec, full_spec]
    else:
        body = _norm_body
        args = (acca, accb, expand, gbias)
        in_specs = [acc_spec, acc_spec,
                    pl.BlockSpec((HHALF, VW), lambda g: (0, 0)),
                    full_spec]
    out = pl.pallas_call(
        body,
        grid=(2,),
        in_specs=in_specs,
        out_specs=pl.BlockSpec((1, NPAD, DIM), lambda g: (g, 0, 0)),
        out_shape=jax.ShapeDtypeStruct((2, NPAD, DIM), jnp.float32),
    )(*args)
    return out.reshape(NT, DIM)


def _pair_body(s_ref, t_ref, w1a_ref, w1b_ref, b1_ref, w2_ref, b2_ref,
               w3_ref, b3_ref, o_ref):
    h = jnp.maximum(
        jnp.dot(s_ref[...], w1a_ref[...], preferred_element_type=jnp.float32)
        + jnp.dot(t_ref[...], w1b_ref[...],
                  preferred_element_type=jnp.float32)
        + b1_ref[...], 0.0)
    h = jnp.maximum(
        jnp.dot(h, w2_ref[...], preferred_element_type=jnp.float32)
        + b2_ref[...], 0.0)
    o_ref[...] = (
        jnp.dot(h, w3_ref[...], preferred_element_type=jnp.float32)
        + b3_ref[...])


def _pair_pallas(xs, xt, w1a, w1b, b1, w2, b2, w3, b3, block_rows):
    rows = xs.shape[0]
    return pl.pallas_call(
        _pair_body,
        grid=(rows // block_rows,),
        in_specs=[
            pl.BlockSpec((block_rows, DIM), lambda i: (i, 0)),
            pl.BlockSpec((block_rows, DIM), lambda i: (i, 0)),
            pl.BlockSpec((DIM, DIM), lambda i: (0, 0)),
            pl.BlockSpec((DIM, DIM), lambda i: (0, 0)),
            pl.BlockSpec((1, DIM), lambda i: (0, 0)),
            pl.BlockSpec((DIM, DIM), lambda i: (0, 0)),
            pl.BlockSpec((1, DIM), lambda i: (0, 0)),
            pl.BlockSpec((DIM, 1), lambda i: (0, 0)),
            pl.BlockSpec((1, 1), lambda i: (0, 0)),
        ],
        out_specs=pl.BlockSpec((block_rows, 1), lambda i: (i, 0)),
        out_shape=jax.ShapeDtypeStruct((rows, 1), jnp.float32),
    )(xs, xt, w1a, w1b, b1, w2, b2, w3, b3)


def _pad_edges(ei, n, goff):
    loop = jnp.arange(n, dtype=ei.dtype)
    s = jnp.concatenate([ei[0], loop])
    d = jnp.concatenate([ei[1], loop])
    etot = s.shape[0]
    ep = -(-etot // (NSUB * CH)) * (NSUB * CH)
    pad = ep - etot
    srcp = jnp.concatenate(
        [s + goff, jnp.full((pad,), goff + PADROW, ei.dtype)])
    dstp = jnp.concatenate(
        [d + goff, jnp.full((pad,), goff + PADROW, ei.dtype)])
    return srcp, dstp


def _gat_layer(h, srcp, dstp, wl, wr, a, expand, gbias, bn_g=None, bn_b=None):
    xla, xlb, xra, xrb = _proj_pallas(h, wl, wr, block_rows=512)
    af = a.reshape(DIM)
    acca = _edge_pass(xla, xra, srcp, dstp, af[:VW])
    accb = _edge_pass(xlb, xrb, srcp, dstp, af[VW:])
    return _norm_pallas(acca, accb, expand, gbias, bn_g=bn_g, bn_b=bn_b)


def kernel(x1, x2, params, edge_index1, edge_index2, jg_edge_index):
    p = params
    n = x1.shape[0]

    zpad = jnp.zeros((NPAD - n, x1.shape[1]), jnp.float32)
    xcat = jnp.concatenate([x1, zpad, x2, zpad], axis=0)

    s1, d1 = _pad_edges(edge_index1, n, 0)
    s2, d2 = _pad_edges(edge_index2, n, NPAD)
    srcp = jnp.stack([s1, s2])
    dstp = jnp.stack([d1, d2])

    w1c = jnp.concatenate([p['fW1'], p['eW1']], axis=1)
    b1c = jnp.concatenate([p['fb1'], p['eb1']])[None, :]
    w2s = jnp.concatenate([p['fW2'], p['eW2']], axis=0)
    b2s = (p['fb2'] + p['eb2'])[None, :]
    h = _mlp2_pallas(xcat, w1c, b1c, w2s, b2s, block_rows=512)

    expand = jnp.repeat(jnp.eye(HHALF, dtype=jnp.float32), DH, axis=1)

    h = _gat_layer(h, srcp, dstp, p['g1Wl'], p['g1Wr'], p['g1a'], expand,
                   p['g1b'][None, :], bn_g=p['bng'][None, :],
                   bn_b=p['bnb'][None, :])
    h = _gat_layer(h, srcp, dstp, p['g2Wl'], p['g2Wr'], p['g2a'], expand,
                   p['g2b'][None, :])

    ej = jg_edge_index.shape[1]
    ejp = -(-ej // (32 * CH)) * (32 * CH)
    i0 = jnp.concatenate([jg_edge_index[0],
                          jnp.zeros((ejp - ej,), jg_edge_index.dtype)])
    i1 = jnp.concatenate([jg_edge_index[1] + NPAD,
                          jnp.full((ejp - ej,), NPAD, jg_edge_index.dtype)])
    xs, xt = _pair_gather(h, i0, i1)
    out = _pair_pallas(xs, xt,
                       p['pW1'][:DIM], p['pW1'][DIM:], p['pb1'][None, :],
                       p['pW2'], p['pb2'][None, :],
                       p['pW3'], p['pb3'][None, :], block_rows=1024)
    return out[:ej]
